# Initial kernel scaffold; baseline (speedup 1.0000x reference)
import jax
import jax.numpy as jnp
from jax.experimental import pallas as pl


def kernel(input, target):
    raise NotImplementedError("write your pallas kernel here")


# speedup vs baseline: 1.4200x; 1.4200x over previous
import jax
import jax.numpy as jnp
from jax.experimental import pallas as pl
from jax.experimental.pallas import tpu as pltpu

_BATCH = 1024
_VOCAB = 100000
_K = 512
_R = 8
_NBLK = _BATCH // _R


def _ce_kernel(x_ref, t_ref, loss_ref):
    x = x_ref[...]
    t = t_ref[0, 0, :]
    m = jnp.max(x, axis=-1)
    s = jnp.sum(jnp.exp(x - m[:, None]), axis=-1)
    logz = m + jnp.log(s)
    col = jax.lax.broadcasted_iota(jnp.int32, x.shape, 1)
    tgt_logit = jnp.sum(jnp.where(col == t[:, None], x, 0.0), axis=-1)
    loss_ref[0, 0, :] = logz - tgt_logit


def _topk_sum_kernel(loss_ref, out_ref):
    losses = loss_ref[...]
    bits = jax.lax.bitcast_convert_type(losses, jnp.int32)

    def body(j, th):
        cand = th | jnp.left_shift(jnp.int32(1), 30 - j)
        cnt = jnp.sum((bits >= cand).astype(jnp.int32))
        return jnp.where(cnt >= _K, cand, th)

    th = jax.lax.fori_loop(0, 31, body, jnp.int32(0))
    kth = jax.lax.bitcast_convert_type(th, jnp.float32)
    gt = bits > th
    cnt_gt = jnp.sum(gt.astype(jnp.int32))
    s_gt = jnp.sum(jnp.where(gt, losses, 0.0))
    out_ref[0, 0] = s_gt + (_K - cnt_gt).astype(jnp.float32) * kth


def kernel(input, target):
    t3 = target.reshape(_NBLK, 1, _R).astype(jnp.int32)
    loss = pl.pallas_call(
        _ce_kernel,
        grid=(_NBLK,),
        in_specs=[
            pl.BlockSpec((_R, _VOCAB), lambda i: (i, 0)),
            pl.BlockSpec((1, 1, _R), lambda i: (i, 0, 0)),
        ],
        out_specs=pl.BlockSpec((1, 1, _R), lambda i: (i, 0, 0)),
        out_shape=jax.ShapeDtypeStruct((_NBLK, 1, _R), jnp.float32),
    )(input, t3)

    out = pl.pallas_call(
        _topk_sum_kernel,
        out_specs=pl.BlockSpec(memory_space=pltpu.SMEM),
        out_shape=jax.ShapeDtypeStruct((1, 1), jnp.float32),
    )(loss.reshape(8, 128))
    return out[0, 0]
